# baseline (speedup 1.0000x reference)
import functools

import numpy as np
import jax
import jax.numpy as jnp
from jax.experimental import pallas as pl
from jax.experimental.pallas import tpu as pltpu

B, C, H, W = 32, 448, 14, 14
HW = H * W
N_BANK, K_IM = 256, 50
OUT_H, OUT_W = 224, 224
SIGMA = 4.0

PSUB = 2
PBLK = 14
NSUB = PBLK // PSUB
NBLK = HW // PBLK
MCHUNK = 7
MSTEPS = HW // MCHUNK


def _resize_matrix(out_n: int, in_n: int) -> np.ndarray:
    s = (np.arange(out_n) + 0.5) * (in_n / out_n) - 0.5
    j = np.arange(in_n)
    w = np.maximum(0.0, 1.0 - np.abs(s[:, None] - j[None, :]))
    return (w / w.sum(axis=1, keepdims=True)).astype(np.float32)


def _blur_matrix(n: int, sigma: float) -> np.ndarray:
    ks = 2 * int(4.0 * sigma + 0.5) + 1
    x = np.arange(ks, dtype=np.float64) - (ks // 2)
    k = np.exp(-(x ** 2) / (2.0 * sigma ** 2))
    k = k / k.sum()
    g = np.zeros((n, n), dtype=np.float64)
    r = ks // 2
    for o in range(n):
        lo = max(0, o - r)
        hi = min(n, o + r + 1)
        g[o, lo:hi] = k[lo - o + r:hi - o + r]
    return g.astype(np.float32)


_M_OP = np.ascontiguousarray(
    _blur_matrix(OUT_H, SIGMA).astype(np.float64)
    @ _resize_matrix(OUT_H, H).astype(np.float64)
).astype(np.float32)


def _fused_kernel(inp_ref, bank_ref, sel_ref, m_ref,
                  pred_ref, amap_ref,
                  pd2_ref, acc_ref, dmin_ref, emb_ref):
    j = pl.program_id(0)

    @pl.when(j == 0)
    def _transpose():
        emb_ref[...] = inp_ref[...].transpose(2, 0, 1).reshape(HW, B, C)

    @pl.when(j < NBLK)
    def _stream():
        row_t = jax.lax.broadcasted_iota(
            jnp.int32, (PSUB * B, PSUB * C), 0) // B
        col_t = jax.lax.broadcasted_iota(
            jnp.int32, (PSUB * B, PSUB * C), 1) // C
        diag = row_t == col_t
        part = jnp.zeros((B, N_BANK), jnp.float32)
        for s in range(NSUB):
            eb = emb_ref[pl.ds(j * PBLK + s * PSUB, PSUB)]
            ebr = eb.reshape(PSUB * B, C)
            tiled = jnp.concatenate([ebr] * PSUB, axis=1)
            lhs = jnp.where(diag, tiled, 0.0)
            bank = bank_ref[:, s * PSUB * C:(s + 1) * PSUB * C]
            dots = jax.lax.dot_general(
                lhs, bank, (((1,), (1,)), ((), ())),
                preferred_element_type=jnp.float32)
            d3 = dots.reshape(PSUB, B, N_BANK)
            e2 = jnp.sum(eb * eb, axis=2)
            b2 = jax.lax.dot_general(
                sel_ref[...], bank * bank, (((1,), (1,)), ((), ())),
                preferred_element_type=jnp.float32)
            pd2 = e2[:, :, None] + b2[:, None, :] - 2.0 * d3
            pd2_ref[pl.ds(j * PBLK + s * PSUB, PSUB)] = pd2
            part = part + jnp.sum(pd2, axis=0)

        @pl.when(j == 0)
        def _init():
            acc_ref[...] = jnp.zeros_like(acc_ref)

        acc_ref[...] += part

    @pl.when(j == NBLK)
    def _finish():
        m_i = jax.lax.broadcasted_iota(jnp.int32, (N_BANK, N_BANK), 0)
        n_i = jax.lax.broadcasted_iota(jnp.int32, (N_BANK, N_BANK), 1)

        def tk_body(i, sel):
            row = acc_ref[pl.ds(i, 1), :]
            col = jnp.transpose(row)
            beats = (col < row) | ((col == row) & (m_i < n_i))
            rank = jnp.sum(beats.astype(jnp.float32), axis=0, keepdims=True)
            selrow = (rank < float(K_IM)).astype(jnp.float32)
            dist = jnp.sqrt(jnp.maximum(row, 1e-12))
            pred = (jnp.sum(dist * selrow, axis=1, keepdims=True)
                    * (1.0 / K_IM))
            pred_ref[pl.ds(i, 1), :] = pred
            row_is_i = jax.lax.broadcasted_iota(
                jnp.int32, (B, N_BANK), 0) == i
            sel = jnp.where(row_is_i, jnp.broadcast_to(selrow, (B, N_BANK)),
                            sel)
            return sel

        sel = jax.lax.fori_loop(
            0, B, tk_body, jnp.zeros((B, N_BANK), jnp.float32))
        keep = sel > 0.5

        def mn_body(i, _):
            pc = pd2_ref[pl.ds(i * MCHUNK, MCHUNK)]
            big = jnp.where(keep[None, :, :], pc, jnp.inf)
            dmin_ref[pl.ds(i, 1)] = jnp.sqrt(
                jnp.maximum(jnp.min(big, axis=2), 1e-12))[None]
            return 0

        jax.lax.fori_loop(0, MSTEPS, mn_body, 0)

        dmin = dmin_ref[...].reshape(HW, B)
        a_iwh = dmin.reshape(H, W, B).transpose(2, 1, 0)
        m_op = m_ref[...]
        t1 = jax.lax.dot_general(
            a_iwh.reshape(B * W, H), m_op, (((1,), (1,)), ((), ())),
            preferred_element_type=jnp.float32)
        t1 = t1.reshape(B, W, OUT_H).transpose(0, 2, 1)
        y = jax.lax.dot_general(
            t1.reshape(B * OUT_H, W), m_op, (((1,), (1,)), ((), ())),
            preferred_element_type=jnp.float32)
        amap_ref[...] = y.reshape(B, OUT_H, OUT_W)


@functools.partial(jax.jit)
def kernel(input_tensor, memory_bank):
    inp = input_tensor.reshape(B, C, HW)
    bank2d = memory_bank.reshape(N_BANK, HW * C)

    sel_np = np.zeros((PSUB, PSUB * C), dtype=np.float32)
    for t in range(PSUB):
        sel_np[t, t * C:(t + 1) * C] = 1.0
    chunk_sel = jnp.asarray(sel_np)

    last = NBLK - 1
    pred2d, amap = pl.pallas_call(
        _fused_kernel,
        grid=(NBLK + 1,),
        in_specs=[
            pl.BlockSpec((B, C, HW), lambda j: (0, 0, 0)),
            pl.BlockSpec((N_BANK, PBLK * C),
                         lambda j: (0, jnp.minimum(j, last))),
            pl.BlockSpec((PSUB, PSUB * C), lambda j: (0, 0)),
            pl.BlockSpec((OUT_H, H), lambda j: (0, 0)),
        ],
        out_specs=[
            pl.BlockSpec((B, 1), lambda j: (0, 0)),
            pl.BlockSpec((B, OUT_H, OUT_W), lambda j: (0, 0, 0)),
        ],
        out_shape=[
            jax.ShapeDtypeStruct((B, 1), jnp.float32),
            jax.ShapeDtypeStruct((B, OUT_H, OUT_W), jnp.float32),
        ],
        scratch_shapes=[
            pltpu.VMEM((HW, B, N_BANK), jnp.float32),
            pltpu.VMEM((B, N_BANK), jnp.float32),
            pltpu.VMEM((MSTEPS, MCHUNK, B), jnp.float32),
            pltpu.VMEM((HW, B, C), jnp.float32),
        ],
    )(inp, bank2d, chunk_sel, jnp.asarray(_M_OP))

    return pred2d.reshape(B), amap[:, None, :, :]


# speedup vs baseline: 1.1849x; 1.1849x over previous
import functools

import numpy as np
import jax
import jax.numpy as jnp
from jax.experimental import pallas as pl
from jax.experimental.pallas import tpu as pltpu

B, C, H, W = 32, 448, 14, 14
HW = H * W
N_BANK, K_IM = 256, 50
OUT_H, OUT_W = 224, 224
SIGMA = 4.0

PSUB = 2
PBLK = 14
NSUB = PBLK // PSUB
NBLK = HW // PBLK
MCHUNK = 7
MSTEPS = HW // MCHUNK


def _resize_matrix(out_n: int, in_n: int) -> np.ndarray:
    s = (np.arange(out_n) + 0.5) * (in_n / out_n) - 0.5
    j = np.arange(in_n)
    w = np.maximum(0.0, 1.0 - np.abs(s[:, None] - j[None, :]))
    return (w / w.sum(axis=1, keepdims=True)).astype(np.float32)


def _blur_matrix(n: int, sigma: float) -> np.ndarray:
    ks = 2 * int(4.0 * sigma + 0.5) + 1
    x = np.arange(ks, dtype=np.float64) - (ks // 2)
    k = np.exp(-(x ** 2) / (2.0 * sigma ** 2))
    k = k / k.sum()
    g = np.zeros((n, n), dtype=np.float64)
    r = ks // 2
    for o in range(n):
        lo = max(0, o - r)
        hi = min(n, o + r + 1)
        g[o, lo:hi] = k[lo - o + r:hi - o + r]
    return g.astype(np.float32)


_M_OP = np.ascontiguousarray(
    _blur_matrix(OUT_H, SIGMA).astype(np.float64)
    @ _resize_matrix(OUT_H, H).astype(np.float64)
).astype(np.float32)


def _fused_kernel(emb_ref, bank_ref, sel_ref, m_ref,
                  pred_ref, amap_ref,
                  pd2_ref, acc_ref, dmin_ref):
    j = pl.program_id(0)

    @pl.when(j < NBLK)
    def _stream():
        row_t = jax.lax.broadcasted_iota(
            jnp.int32, (PSUB * B, PSUB * C), 0) // B
        col_t = jax.lax.broadcasted_iota(
            jnp.int32, (PSUB * B, PSUB * C), 1) // C
        diag = row_t == col_t
        part = jnp.zeros((B, N_BANK), jnp.float32)
        for s in range(NSUB):
            eb = emb_ref[s * PSUB:(s + 1) * PSUB]
            ebr = eb.reshape(PSUB * B, C)
            tiled = jnp.concatenate([ebr] * PSUB, axis=1)
            lhs = jnp.where(diag, tiled, 0.0)
            bank = bank_ref[:, s * PSUB * C:(s + 1) * PSUB * C]
            dots = jax.lax.dot_general(
                lhs, bank, (((1,), (1,)), ((), ())),
                preferred_element_type=jnp.float32)
            d3 = dots.reshape(PSUB, B, N_BANK)
            e2 = jnp.sum(eb * eb, axis=2)
            b2 = jax.lax.dot_general(
                sel_ref[...], bank * bank, (((1,), (1,)), ((), ())),
                preferred_element_type=jnp.float32)
            pd2 = e2[:, :, None] + b2[:, None, :] - 2.0 * d3
            pd2_ref[pl.ds(j * PBLK + s * PSUB, PSUB)] = pd2
            part = part + jnp.sum(pd2, axis=0)

        @pl.when(j == 0)
        def _init():
            acc_ref[...] = jnp.zeros_like(acc_ref)

        acc_ref[...] += part

    @pl.when(j == NBLK)
    def _finish():
        m_i = jax.lax.broadcasted_iota(jnp.int32, (N_BANK, N_BANK), 0)
        n_i = jax.lax.broadcasted_iota(jnp.int32, (N_BANK, N_BANK), 1)

        def tk_body(i, sel):
            row = acc_ref[pl.ds(i, 1), :]
            col = jnp.transpose(row)
            beats = (col < row) | ((col == row) & (m_i < n_i))
            rank = jnp.sum(beats.astype(jnp.float32), axis=0, keepdims=True)
            selrow = (rank < float(K_IM)).astype(jnp.float32)
            dist = jnp.sqrt(jnp.maximum(row, 1e-12))
            pred = (jnp.sum(dist * selrow, axis=1, keepdims=True)
                    * (1.0 / K_IM))
            pred_ref[pl.ds(i, 1), :] = pred
            row_is_i = jax.lax.broadcasted_iota(
                jnp.int32, (B, N_BANK), 0) == i
            sel = jnp.where(row_is_i, jnp.broadcast_to(selrow, (B, N_BANK)),
                            sel)
            return sel

        sel = jax.lax.fori_loop(
            0, B, tk_body, jnp.zeros((B, N_BANK), jnp.float32))
        keep = sel > 0.5

        def mn_body(i, _):
            pc = pd2_ref[pl.ds(i * MCHUNK, MCHUNK)]
            big = jnp.where(keep[None, :, :], pc, jnp.inf)
            dmin_ref[pl.ds(i, 1)] = jnp.sqrt(
                jnp.maximum(jnp.min(big, axis=2), 1e-12))[None]
            return 0

        jax.lax.fori_loop(0, MSTEPS, mn_body, 0)

        dmin = dmin_ref[...].reshape(HW, B)
        a_iwh = dmin.reshape(H, W, B).transpose(2, 1, 0)
        m_op = m_ref[...]
        t1 = jax.lax.dot_general(
            a_iwh.reshape(B * W, H), m_op, (((1,), (1,)), ((), ())),
            preferred_element_type=jnp.float32)
        t1 = t1.reshape(B, W, OUT_H).transpose(0, 2, 1)
        y = jax.lax.dot_general(
            t1.reshape(B * OUT_H, W), m_op, (((1,), (1,)), ((), ())),
            preferred_element_type=jnp.float32)
        amap_ref[...] = y.reshape(B, OUT_H, OUT_W)


@functools.partial(jax.jit)
def kernel(input_tensor, memory_bank):
    emb = jnp.transpose(input_tensor, (2, 3, 0, 1)).reshape(HW, B, C)
    bank2d = memory_bank.reshape(N_BANK, HW * C)

    sel_np = np.zeros((PSUB, PSUB * C), dtype=np.float32)
    for t in range(PSUB):
        sel_np[t, t * C:(t + 1) * C] = 1.0
    chunk_sel = jnp.asarray(sel_np)

    last = NBLK - 1
    pred2d, amap = pl.pallas_call(
        _fused_kernel,
        grid=(NBLK + 1,),
        in_specs=[
            pl.BlockSpec((PBLK, B, C),
                         lambda j: (jnp.minimum(j, last), 0, 0)),
            pl.BlockSpec((N_BANK, PBLK * C),
                         lambda j: (0, jnp.minimum(j, last))),
            pl.BlockSpec((PSUB, PSUB * C), lambda j: (0, 0)),
            pl.BlockSpec((OUT_H, H), lambda j: (0, 0)),
        ],
        out_specs=[
            pl.BlockSpec((B, 1), lambda j: (0, 0)),
            pl.BlockSpec((B, OUT_H, OUT_W), lambda j: (0, 0, 0)),
        ],
        out_shape=[
            jax.ShapeDtypeStruct((B, 1), jnp.float32),
            jax.ShapeDtypeStruct((B, OUT_H, OUT_W), jnp.float32),
        ],
        scratch_shapes=[
            pltpu.VMEM((HW, B, N_BANK), jnp.float32),
            pltpu.VMEM((B, N_BANK), jnp.float32),
            pltpu.VMEM((MSTEPS, MCHUNK, B), jnp.float32),
        ],
    )(emb, bank2d, chunk_sel, jnp.asarray(_M_OP))

    return pred2d.reshape(B), amap[:, None, :, :]
